# baseline (speedup 1.0000x reference)
import functools

import jax
import jax.numpy as jnp
from jax import lax
from jax.experimental import pallas as pl
from jax.experimental.pallas import tpu as pltpu
from jax.experimental.pallas import tpu_sc as plsc

_B, _S, _A, _D, _C, _K = 16, 512, 8, 512, 1024, 34
_SA = _S * _A
_NW = 32
_ROWS_W = (_B * _C) // _NW
_CHUNK = 64
_NCHUNK = _ROWS_W // _CHUNK
_ROWS_TC = 8192
_KP = 128


def _gather_body(batch0, nrows, wr_hbm, idx_hbm, out_hbm,
                 idx_v, buf0, buf1, sem0, sem1):
    wid = lax.axis_index("s") * 2 + lax.axis_index("c")
    rows_w = nrows // _NW
    nchunk = rows_w // _CHUNK
    wpb = _C // rows_w
    base_row = wid * rows_w
    row_off = (batch0 + wid // wpb) * _SA
    pltpu.sync_copy(idx_hbm.at[pl.ds(base_row, rows_w)], idx_v)
    for j in range(rows_w // 16):
        sl = pl.ds(j * 16, 16)
        idx_v[sl] = idx_v[sl] + row_off
    bufs = (buf0, buf1)
    sems = (sem0, sem1)
    copies = [
        pltpu.make_async_copy(
            wr_hbm.at[idx_v.at[pl.ds(g * _CHUNK, _CHUNK)]],
            bufs[g % 2], sems[g % 2])
        for g in range(nchunk)
    ]
    copies[0].start()
    for g in range(nchunk):
        if g + 1 < nchunk:
            copies[g + 1].start()
        copies[g].wait()
        pltpu.sync_copy(bufs[g % 2],
                        out_hbm.at[pl.ds(base_row + g * _CHUNK, _CHUNK)])


def _sc_gather(wr_flat, cidx_half, batch0, nrows):
    mesh = plsc.VectorSubcoreMesh(core_axis_name="c", subcore_axis_name="s")
    kern = functools.partial(
        pl.kernel, functools.partial(_gather_body, batch0, nrows), mesh=mesh,
        out_type=jax.ShapeDtypeStruct((nrows, _D), jnp.float32),
        scratch_types=[
            pltpu.VMEM((nrows // _NW,), jnp.int32),
            pltpu.VMEM((_CHUNK, _D), jnp.float32),
            pltpu.VMEM((_CHUNK, _D), jnp.float32),
            pltpu.SemaphoreType.DMA,
            pltpu.SemaphoreType.DMA,
        ],
    )()
    return kern(wr_flat, cidx_half)


def _classify_body(x_ref, wt_ref, bias_ref, red_ref, lab_ref,
                   loss_ref, bcp_ref):
    i = pl.program_id(0)
    x = x_ref[...]
    logits = jnp.dot(x, wt_ref[...], preferred_element_type=jnp.float32)
    lg = logits + bias_ref[...]
    m = jnp.max(lg, axis=-1, keepdims=True)
    e = jnp.exp(lg - m)
    ind = (lg >= m).astype(jnp.float32)
    nb = _ROWS_TC // _C
    kio3 = lax.broadcasted_iota(jnp.int32, (nb, _C, _KP), 2)
    lab3 = lab_ref[...].reshape(nb, _C, 1)
    sel = jnp.where(kio3 == lab3, lg.reshape(nb, _C, _KP),
                    0.0).reshape(_ROWS_TC, _KP)
    es = jnp.dot(e, red_ref[...], preferred_element_type=jnp.float32)
    inds = jnp.dot(ind, red_ref[...], preferred_element_type=jnp.float32)
    lls = jnp.dot(sel, red_ref[...], preferred_element_type=jnp.float32)
    logz = m + jnp.log(es[:, 0:1])
    ll = lls[:, 0:1]
    eb = lax.shift_right_logical(
        lax.bitcast_convert_type(inds[:, 1:2], jnp.int32), 23)
    bcp_ref[...] = lax.div(127 - eb, 2)

    @pl.when(i == 0)
    def _():
        loss_ref[...] = jnp.zeros_like(loss_ref)

    loss_ref[...] += jnp.sum(logz - ll, keepdims=True)


def _tc_classify(cand_half, wt, bias, red, lab2):
    nrows = cand_half.shape[0]
    grid = nrows // _ROWS_TC
    return pl.pallas_call(
        _classify_body,
        grid=(grid,),
        in_specs=[
            pl.BlockSpec((_ROWS_TC, _D), lambda i: (i, 0)),
            pl.BlockSpec((_D, _KP), lambda i: (0, 0)),
            pl.BlockSpec((1, _KP), lambda i: (0, 0)),
            pl.BlockSpec((_KP, _KP), lambda i: (0, 0)),
            pl.BlockSpec((_ROWS_TC // _C, _C), lambda i: (i, 0)),
        ],
        out_specs=[
            pl.BlockSpec((1, 1), lambda i: (0, 0)),
            pl.BlockSpec((_ROWS_TC, 1), lambda i: (i, 0)),
        ],
        out_shape=[
            jax.ShapeDtypeStruct((1, 1), jnp.float32),
            jax.ShapeDtypeStruct((nrows, 1), jnp.int32),
        ],
    )(cand_half, wt, bias, red, lab2)


def _predict_body(idx_hbm, bcp_hbm, out_hbm, mask_hbm, idx_v, val_v, table,
                  mbuf):
    wid = lax.axis_index("s") * 2 + lax.axis_index("c")

    @pl.when(wid < _B)
    def _():
        pltpu.sync_copy(idx_hbm.at[wid], idx_v.at[pl.ds(0, _C)])
        pltpu.sync_copy(bcp_hbm.at[wid], val_v)
        zv = jnp.zeros((16,), jnp.int32)

        def zbody(t, carry):
            table[pl.ds(t * 16, 16)] = zv
            return carry

        lax.fori_loop(0, _SA // 16, zbody, 0)
        iota = lax.iota(jnp.int32, 16)

        def gbody(g, carry):
            base = g * 16
            sl = pl.ds(base, 16)
            iv = idx_v[sl]
            vv = val_v[sl]
            mbuf[sl] = jnp.where(iv != 0, 1.0, 0.0).astype(jnp.float32)
            keep = jnp.ones((16,), jnp.bool_)
            for s in range(1, 16):
                shifted = idx_v[pl.ds(base + s, 16)]
                dup = (iv == shifted) & (iota < (16 - s))
                keep = keep & jnp.logical_not(dup)
            plsc.store_scatter(table, [iv], vv, mask=keep)
            return carry

        lax.fori_loop(0, _C // 16, gbody, 0)
        pltpu.sync_copy(table, out_hbm.at[wid])
        pltpu.sync_copy(mbuf, mask_hbm.at[wid])


def _sc_predict(cidx, bcp):
    mesh = plsc.VectorSubcoreMesh(core_axis_name="c", subcore_axis_name="s")
    kern = functools.partial(
        pl.kernel, _predict_body, mesh=mesh,
        out_type=[
            jax.ShapeDtypeStruct((_B, _SA), jnp.int32),
            jax.ShapeDtypeStruct((_B, _C), jnp.float32),
        ],
        scratch_types=[
            pltpu.VMEM((_C + 16,), jnp.int32),
            pltpu.VMEM((_C,), jnp.int32),
            pltpu.VMEM((_SA,), jnp.int32),
            pltpu.VMEM((_C,), jnp.float32),
        ],
        compiler_params=pltpu.CompilerParams(needs_layout_passes=False),
    )()
    return kern(cidx, bcp)


def kernel(word_mask, word_repr, candidates_idx, candidate_label, anchor_loc,
           anchor_label, anchor_cls, batch_candidate_num, key_candidates,
           key_candidate_mask, key_candidate_len, key_candidate_loc, W, b):
    wr_flat = word_repr.reshape(_B * _SA, _D)
    cidx_flat = candidates_idx.reshape(-1)

    wt = jnp.zeros((_D, _KP), jnp.float32).at[:, :_K].set(W.T)
    bias = jnp.full((1, _KP), -1e30, jnp.float32).at[0, :_K].set(b)
    kar = jnp.arange(_KP, dtype=jnp.float32)
    red = jnp.zeros((_KP, _KP), jnp.float32)
    red = red.at[:, 0].set(1.0).at[:, 1].set(jnp.exp2(-2.0 * kar))

    cand_flat = _sc_gather(wr_flat, cidx_flat, 0, _B * _C)
    loss0, bcp2 = _tc_classify(cand_flat, wt, bias, red, candidate_label)

    loss = loss0[0, 0] / float(_B * _C)
    bcp = bcp2.reshape(_B, _C)
    pred, mask = _sc_predict(candidates_idx, bcp)

    return (loss, pred.reshape(_B, _S, _A),
            cand_flat.reshape(_B, _C, _D), candidate_label,
            bcp, mask)


# speedup vs baseline: 1.0172x; 1.0172x over previous
import functools

import jax
import jax.numpy as jnp
from jax import lax
from jax.experimental import pallas as pl
from jax.experimental.pallas import tpu as pltpu
from jax.experimental.pallas import tpu_sc as plsc

_B, _S, _A, _D, _C, _K = 16, 512, 8, 512, 1024, 34
_SA = _S * _A
_NW = 32
_ROWS_W = (_B * _C) // _NW
_CHUNK = 64
_NCHUNK = _ROWS_W // _CHUNK
_ROWS_TC = 4096
_KP = 128


def _gather_body(batch0, nrows, wr_hbm, idx_hbm, out_hbm,
                 idx_v, buf0, buf1, sem0, sem1):
    wid = lax.axis_index("s") * 2 + lax.axis_index("c")
    rows_w = nrows // _NW
    nchunk = rows_w // _CHUNK
    wpb = _C // rows_w
    base_row = wid * rows_w
    row_off = (batch0 + wid // wpb) * _SA
    pltpu.sync_copy(idx_hbm.at[pl.ds(base_row, rows_w)], idx_v)
    for j in range(rows_w // 16):
        sl = pl.ds(j * 16, 16)
        idx_v[sl] = idx_v[sl] + row_off
    bufs = (buf0, buf1)
    sems = (sem0, sem1)
    copies = [
        pltpu.make_async_copy(
            wr_hbm.at[idx_v.at[pl.ds(g * _CHUNK, _CHUNK)]],
            bufs[g % 2], sems[g % 2])
        for g in range(nchunk)
    ]
    copies[0].start()
    for g in range(nchunk):
        if g + 1 < nchunk:
            copies[g + 1].start()
        copies[g].wait()
        pltpu.sync_copy(bufs[g % 2],
                        out_hbm.at[pl.ds(base_row + g * _CHUNK, _CHUNK)])


def _sc_gather(wr_flat, cidx_half, batch0, nrows):
    mesh = plsc.VectorSubcoreMesh(core_axis_name="c", subcore_axis_name="s")
    kern = functools.partial(
        pl.kernel, functools.partial(_gather_body, batch0, nrows), mesh=mesh,
        out_type=jax.ShapeDtypeStruct((nrows, _D), jnp.float32),
        scratch_types=[
            pltpu.VMEM((nrows // _NW,), jnp.int32),
            pltpu.VMEM((_CHUNK, _D), jnp.float32),
            pltpu.VMEM((_CHUNK, _D), jnp.float32),
            pltpu.SemaphoreType.DMA,
            pltpu.SemaphoreType.DMA,
        ],
    )()
    return kern(wr_flat, cidx_half)


def _classify_body(x_ref, wt_ref, bias_ref, red_ref, lab_ref,
                   loss_ref, bcp_ref):
    i = pl.program_id(0)
    x = x_ref[...]
    logits = jnp.dot(x, wt_ref[...], preferred_element_type=jnp.float32)
    lg = logits + bias_ref[...]
    kio = lax.broadcasted_iota(jnp.int32, (_ROWS_TC, _KP), 1)
    m = jnp.max(lg, axis=-1, keepdims=True)
    e = jnp.exp(lg - m)
    ind = (lg >= m).astype(jnp.float32)
    sel = jnp.where(kio == lab_ref[...], lg, 0.0)
    es = jnp.dot(e, red_ref[...], preferred_element_type=jnp.float32)
    inds = jnp.dot(ind, red_ref[...], preferred_element_type=jnp.float32)
    lls = jnp.dot(sel, red_ref[...], preferred_element_type=jnp.float32)
    logz = m + jnp.log(es[:, 0:1])
    ll = lls[:, 0:1]
    eb = lax.shift_right_logical(
        lax.bitcast_convert_type(inds[:, 1:2], jnp.int32), 23)
    bcp_ref[...] = lax.div(127 - eb, 2)

    @pl.when(i == 0)
    def _():
        loss_ref[...] = jnp.zeros_like(loss_ref)

    loss_ref[...] += jnp.sum(logz - ll, keepdims=True)


def _tc_classify(cand_half, wt, bias, red, lab2):
    nrows = cand_half.shape[0]
    grid = nrows // _ROWS_TC
    return pl.pallas_call(
        _classify_body,
        grid=(grid,),
        in_specs=[
            pl.BlockSpec((_ROWS_TC, _D), lambda i: (i, 0)),
            pl.BlockSpec((_D, _KP), lambda i: (0, 0)),
            pl.BlockSpec((1, _KP), lambda i: (0, 0)),
            pl.BlockSpec((_KP, _KP), lambda i: (0, 0)),
            pl.BlockSpec((_ROWS_TC, 1), lambda i: (i, 0)),
        ],
        out_specs=[
            pl.BlockSpec((1, 1), lambda i: (0, 0)),
            pl.BlockSpec((_ROWS_TC, 1), lambda i: (i, 0)),
        ],
        out_shape=[
            jax.ShapeDtypeStruct((1, 1), jnp.float32),
            jax.ShapeDtypeStruct((nrows, 1), jnp.int32),
        ],
    )(cand_half, wt, bias, red, lab2)


def _predict_body(idx_hbm, bcp_hbm, out_hbm, mask_hbm, idx_v, val_v, table,
                  mbuf):
    wid = lax.axis_index("s") * 2 + lax.axis_index("c")

    @pl.when(wid < _B)
    def _():
        pltpu.sync_copy(idx_hbm.at[wid], idx_v.at[pl.ds(0, _C)])
        pltpu.sync_copy(bcp_hbm.at[wid], val_v)
        zv = jnp.zeros((16,), jnp.int32)

        def zbody(t, carry):
            table[pl.ds(t * 16, 16)] = zv
            return carry

        lax.fori_loop(0, _SA // 16, zbody, 0)
        iota = lax.iota(jnp.int32, 16)

        def gbody(g, carry):
            base = g * 16
            sl = pl.ds(base, 16)
            iv = idx_v[sl]
            vv = val_v[sl]
            mbuf[sl] = jnp.where(iv != 0, 1.0, 0.0).astype(jnp.float32)
            keep = jnp.ones((16,), jnp.bool_)
            for s in range(1, 16):
                shifted = idx_v[pl.ds(base + s, 16)]
                dup = (iv == shifted) & (iota < (16 - s))
                keep = keep & jnp.logical_not(dup)
            plsc.store_scatter(table, [iv], vv, mask=keep)
            return carry

        lax.fori_loop(0, _C // 16, gbody, 0)
        pltpu.sync_copy(table, out_hbm.at[wid])
        pltpu.sync_copy(mbuf, mask_hbm.at[wid])


def _sc_predict(cidx, bcp):
    mesh = plsc.VectorSubcoreMesh(core_axis_name="c", subcore_axis_name="s")
    kern = functools.partial(
        pl.kernel, _predict_body, mesh=mesh,
        out_type=[
            jax.ShapeDtypeStruct((_B, _SA), jnp.int32),
            jax.ShapeDtypeStruct((_B, _C), jnp.float32),
        ],
        scratch_types=[
            pltpu.VMEM((_C + 16,), jnp.int32),
            pltpu.VMEM((_C,), jnp.int32),
            pltpu.VMEM((_SA,), jnp.int32),
            pltpu.VMEM((_C,), jnp.float32),
        ],
        compiler_params=pltpu.CompilerParams(needs_layout_passes=False),
    )()
    return kern(cidx, bcp)


def kernel(word_mask, word_repr, candidates_idx, candidate_label, anchor_loc,
           anchor_label, anchor_cls, batch_candidate_num, key_candidates,
           key_candidate_mask, key_candidate_len, key_candidate_loc, W, b):
    wr_flat = word_repr.reshape(_B * _SA, _D)
    cidx_flat = candidates_idx.reshape(-1)

    wt = jnp.zeros((_D, _KP), jnp.float32).at[:, :_K].set(W.T)
    bias = jnp.full((1, _KP), -1e30, jnp.float32).at[0, :_K].set(b)
    kar = jnp.arange(_KP, dtype=jnp.float32)
    red = jnp.zeros((_KP, _KP), jnp.float32)
    red = red.at[:, 0].set(1.0).at[:, 1].set(jnp.exp2(-2.0 * kar))
    lab_flat = candidate_label.reshape(_B * _C, 1)

    cand_flat = _sc_gather(wr_flat, cidx_flat, 0, _B * _C)
    loss0, bcp2 = _tc_classify(cand_flat, wt, bias, red, lab_flat)

    loss = loss0[0, 0] / float(_B * _C)
    bcp = bcp2.reshape(_B, _C)
    pred, mask = _sc_predict(candidates_idx, bcp)

    return (loss, pred.reshape(_B, _S, _A),
            cand_flat.reshape(_B, _C, _D), candidate_label,
            bcp, mask)
